# baseline (speedup 1.0000x reference)
import functools

import jax
import jax.numpy as jnp
from jax import lax
from jax.experimental import pallas as pl
from jax.experimental.pallas import tpu as pltpu
from jax.experimental.pallas import tpu_sc as plsc

NC = 2
NS = 16
NW = NC * NS
EB = 80
CH = 50
NBUF = 3
ZB = 80


def _sc_agg(src_hbm, dst_hbm, x_hbm, out_hbm, sidx, didx, rows, aggsh,
            *sems, n_nodes, nch):
    sg = sems[:NBUF]
    ss = sems[NBUF:]
    c = lax.axis_index("c")
    s = lax.axis_index("s")
    wid = c * NS + s

    d = rows.shape[2]
    zvec = jnp.zeros((16,), jnp.float32)

    @pl.loop(0, ZB)
    def _zero_row(r):
        for cc in range(d // 16):
            rows[0, r, pl.ds(cc * 16, 16)] = zvec

    @pl.loop(s, n_nodes // ZB, step=NS)
    def _zero_chunk(j):
        pltpu.sync_copy(rows.at[0, pl.ds(0, ZB)],
                        aggsh.at[pl.ds(j * ZB, ZB)])

    plsc.subcore_barrier()

    pd = NBUF - 1

    @pl.loop(0, nch)
    def _chunk(g):
        pltpu.sync_copy(src_hbm.at[wid, g], sidx)
        pltpu.sync_copy(dst_hbm.at[wid, g], didx)
        for b in range(pd):
            pltpu.async_copy(x_hbm.at[sidx.at[b]], rows.at[b], sg[b])

        @pl.loop(0, CH, step=NBUF)
        def _blk(j):
            def slot(u):
                b = u % NBUF
                bp = (u + pd) % NBUF
                k = j + u

                def body():
                    pltpu.make_async_copy(x_hbm.at[sidx.at[k]], rows.at[b],
                                          sg[b]).wait()
                    pltpu.async_copy(rows.at[b], aggsh.at[didx.at[k]], ss[b],
                                     add=True)

                    @pl.when(k + pd < CH)
                    def _prefetch():
                        def wait_prev():
                            pltpu.make_async_copy(
                                rows.at[bp], aggsh.at[didx.at[k - 1]],
                                ss[bp]).wait()

                        if u == 0:
                            pl.when(k > 0)(wait_prev)
                        else:
                            wait_prev()
                        pltpu.async_copy(x_hbm.at[sidx.at[k + pd]],
                                         rows.at[bp], sg[bp])

                if u == 0:
                    body()
                else:
                    pl.when(j + u < CH)(body)

            for u in range(NBUF):
                slot(u)

        for m in range(CH - NBUF, CH):
            pltpu.make_async_copy(rows.at[m % NBUF], aggsh.at[didx.at[m]],
                                  ss[m % NBUF]).wait()

    plsc.subcore_barrier()

    @pl.loop(s, n_nodes // ZB, step=NS)
    def _writeout(j):
        pltpu.sync_copy(aggsh.at[pl.ds(j * ZB, ZB)],
                        out_hbm.at[pl.ds(c * n_nodes + j * ZB, ZB)])


def _dense_body(x_ref, a_ref, w1_ref, b1_ref, w2_ref, b2_ref, wl_ref, bl_ref,
                o_ref):
    xb = x_ref[...]
    h = xb + a_ref[0] + a_ref[1]
    h = jnp.dot(h, w1_ref[...], preferred_element_type=jnp.float32)
    h = jnp.maximum(h + b1_ref[...], 0.0)
    o = jnp.dot(h, w2_ref[...], preferred_element_type=jnp.float32) + b2_ref[...]
    o = o + jnp.dot(xb, wl_ref[...], preferred_element_type=jnp.float32)
    o_ref[...] = o + bl_ref[...]


def kernel(x, edge_index, edge_weights, edge_attr, with_linear,
           W1, b1, W2, b2, Wlin, blin):
    n, d = x.shape
    e = edge_index.shape[1]
    blk = NW * EB
    nb = -(-e // blk)
    nch = -(-nb // CH)
    nb = nch * CH
    e_pad = nb * blk

    if e_pad == e:
        src, dst = edge_index[0], edge_index[1]
    else:
        src = jnp.concatenate(
            [edge_index[0], jnp.zeros((e_pad - e,), jnp.int32)])
        dst = jnp.concatenate(
            [edge_index[1], jnp.full((e_pad - e,), n, jnp.int32)])
    src = src.reshape(NW, nch, CH, EB)
    dst = dst.reshape(NW, nch, CH, EB)

    sc_call = pl.kernel(
        functools.partial(_sc_agg, n_nodes=n, nch=nch),
        out_type=jax.ShapeDtypeStruct((NC * n, d), jnp.float32),
        mesh=plsc.VectorSubcoreMesh(core_axis_name="c", subcore_axis_name="s",
                                    num_cores=NC, num_subcores=NS),
        scratch_types=[
            pltpu.VMEM((CH, EB), jnp.int32),
            pltpu.VMEM((CH, EB), jnp.int32),
            pltpu.VMEM((NBUF, EB, d), jnp.float32),
            pltpu.VMEM_SHARED((n + 8, d), jnp.float32),
        ] + [pltpu.SemaphoreType.DMA] * (2 * NBUF),
    )
    agg2 = sc_call(src, dst, x).reshape(NC, n, d)

    gate = jnp.where(jnp.asarray(with_linear) != 0, jnp.float32(1.0),
                     jnp.float32(0.0))
    wl = Wlin * gate
    bl = (blin * gate).reshape(1, d)
    b1r = b1.reshape(1, d)
    b2r = b2.reshape(1, d)

    bn = 2000
    grid = (n // bn,)
    out = pl.pallas_call(
        _dense_body,
        grid=grid,
        in_specs=[
            pl.BlockSpec((bn, d), lambda i: (i, 0)),
            pl.BlockSpec((NC, bn, d), lambda i: (0, i, 0)),
            pl.BlockSpec((d, d), lambda i: (0, 0)),
            pl.BlockSpec((1, d), lambda i: (0, 0)),
            pl.BlockSpec((d, d), lambda i: (0, 0)),
            pl.BlockSpec((1, d), lambda i: (0, 0)),
            pl.BlockSpec((d, d), lambda i: (0, 0)),
            pl.BlockSpec((1, d), lambda i: (0, 0)),
        ],
        out_specs=pl.BlockSpec((bn, d), lambda i: (i, 0)),
        out_shape=jax.ShapeDtypeStruct((n, d), jnp.float32),
    )(x, agg2, W1, b1r, W2, b2r, wl, bl)
    return out


# speedup vs baseline: 16.9465x; 16.9465x over previous
import functools

import jax
import jax.numpy as jnp
from jax import lax
from jax.experimental import pallas as pl
from jax.experimental.pallas import tpu as pltpu
from jax.experimental.pallas import tpu_sc as plsc

NC = 2
NS = 16
NW = NC * NS
EB = 80
CH = 25
NBUF = 3
ZB = 80


def _sc_agg(src_hbm, dst_hbm, x_hbm, out_hbm, sidx, didx, rows, aggsh,
            *sems, n_nodes, nch):
    sg = sems[:NBUF]
    ss = sems[NBUF:]
    c = lax.axis_index("c")
    s = lax.axis_index("s")
    wid = c * NS + s

    d = rows.shape[2]
    zvec = jnp.zeros((16,), jnp.float32)

    @pl.loop(0, ZB)
    def _zero_row(r):
        for cc in range(d // 16):
            rows[0, r, pl.ds(cc * 16, 16)] = zvec

    @pl.loop(s, n_nodes // ZB, step=NS)
    def _zero_chunk(j):
        pltpu.sync_copy(rows.at[0, pl.ds(0, ZB)],
                        aggsh.at[pl.ds(j * ZB, ZB)])

    plsc.subcore_barrier()

    pd = NBUF - 1

    @pl.loop(0, nch)
    def _chunk(g):
        pltpu.sync_copy(src_hbm.at[wid, g], sidx)
        pltpu.sync_copy(dst_hbm.at[wid, g], didx)
        for b in range(pd):
            pltpu.async_copy(x_hbm.at[sidx.at[b]], rows.at[b], sg[b])

        @pl.loop(0, CH, step=NBUF)
        def _blk(j):
            def slot(u):
                b = u % NBUF
                bp = (u + pd) % NBUF
                k = j + u

                def body():
                    pltpu.make_async_copy(x_hbm.at[sidx.at[k]], rows.at[b],
                                          sg[b]).wait()
                    pltpu.async_copy(rows.at[b], aggsh.at[didx.at[k]], ss[b],
                                     add=True)

                    @pl.when(k + pd < CH)
                    def _prefetch():
                        def wait_prev():
                            pltpu.make_async_copy(
                                rows.at[bp], aggsh.at[didx.at[k - 1]],
                                ss[bp]).wait()

                        if u == 0:
                            pl.when(k > 0)(wait_prev)
                        else:
                            wait_prev()
                        pltpu.async_copy(x_hbm.at[sidx.at[k + pd]],
                                         rows.at[bp], sg[bp])

                if u == 0:
                    body()
                else:
                    pl.when(j + u < CH)(body)

            for u in range(NBUF):
                slot(u)

        for m in range(CH - NBUF, CH):
            pltpu.make_async_copy(rows.at[m % NBUF], aggsh.at[didx.at[m]],
                                  ss[m % NBUF]).wait()

    plsc.subcore_barrier()

    @pl.loop(s, n_nodes // ZB, step=NS)
    def _writeout(j):
        pltpu.sync_copy(aggsh.at[pl.ds(j * ZB, ZB)],
                        out_hbm.at[pl.ds(c * n_nodes + j * ZB, ZB)])


def _dense_body(x_ref, a_ref, w1_ref, b1_ref, w2_ref, b2_ref, wl_ref, bl_ref,
                o_ref):
    xb = x_ref[...]
    h = xb + a_ref[0] + a_ref[1]
    h = jnp.dot(h, w1_ref[...], preferred_element_type=jnp.float32)
    h = jnp.maximum(h + b1_ref[...], 0.0)
    o = jnp.dot(h, w2_ref[...], preferred_element_type=jnp.float32) + b2_ref[...]
    o = o + jnp.dot(xb, wl_ref[...], preferred_element_type=jnp.float32)
    o_ref[...] = o + bl_ref[...]


def kernel(x, edge_index, edge_weights, edge_attr, with_linear,
           W1, b1, W2, b2, Wlin, blin):
    n, d = x.shape
    e = edge_index.shape[1]
    blk = NW * EB
    nb = -(-e // blk)
    nch = -(-nb // CH)
    nb = nch * CH
    e_pad = nb * blk

    if e_pad == e:
        src, dst = edge_index[0], edge_index[1]
    else:
        src = jnp.concatenate(
            [edge_index[0], jnp.zeros((e_pad - e,), jnp.int32)])
        dst = jnp.concatenate(
            [edge_index[1], jnp.full((e_pad - e,), n, jnp.int32)])
    src = src.reshape(NW, nch, CH, EB)
    dst = dst.reshape(NW, nch, CH, EB)

    sc_call = pl.kernel(
        functools.partial(_sc_agg, n_nodes=n, nch=nch),
        out_type=jax.ShapeDtypeStruct((NC * n, d), jnp.float32),
        mesh=plsc.VectorSubcoreMesh(core_axis_name="c", subcore_axis_name="s",
                                    num_cores=NC, num_subcores=NS),
        scratch_types=[
            pltpu.VMEM((CH, EB), jnp.int32),
            pltpu.VMEM((CH, EB), jnp.int32),
            pltpu.VMEM((NBUF, EB, d), jnp.float32),
            pltpu.VMEM_SHARED((n + 8, d), jnp.float32),
        ] + [pltpu.SemaphoreType.DMA] * (2 * NBUF),
    )
    agg2 = sc_call(src, dst, x).reshape(NC, n, d)

    gate = jnp.where(jnp.asarray(with_linear) != 0, jnp.float32(1.0),
                     jnp.float32(0.0))
    wl = Wlin * gate
    bl = (blin * gate).reshape(1, d)
    b1r = b1.reshape(1, d)
    b2r = b2.reshape(1, d)

    bn = 2000
    grid = (n // bn,)
    out = pl.pallas_call(
        _dense_body,
        grid=grid,
        in_specs=[
            pl.BlockSpec((bn, d), lambda i: (i, 0)),
            pl.BlockSpec((NC, bn, d), lambda i: (0, i, 0)),
            pl.BlockSpec((d, d), lambda i: (0, 0)),
            pl.BlockSpec((1, d), lambda i: (0, 0)),
            pl.BlockSpec((d, d), lambda i: (0, 0)),
            pl.BlockSpec((1, d), lambda i: (0, 0)),
            pl.BlockSpec((d, d), lambda i: (0, 0)),
            pl.BlockSpec((1, d), lambda i: (0, 0)),
        ],
        out_specs=pl.BlockSpec((bn, d), lambda i: (i, 0)),
        out_shape=jax.ShapeDtypeStruct((n, d), jnp.float32),
    )(x, agg2, W1, b1r, W2, b2r, wl, bl)
    return out
